# baseline (speedup 1.0000x reference)
import jax
import jax.numpy as jnp
from jax.experimental import pallas as pl
from jax.experimental.pallas import tpu as pltpu

_N, _M, _D = 10000, 64, 32
_BN = 500
_NB = _N // _BN
_PACK = 8
_KP = _PACK * _D
_NG = _M // _PACK

_F32 = jnp.float32
_BF16 = jnp.bfloat16
_CONTRACT0 = (((0,), (0,)), ((), ()))


def _fused_kernel(x0_ref, b_ref, w1_ref, w2_ref, wm1_ref, bm1_ref,
                  wm2_ref, bm2_ref, wm3_ref, bm3_ref, out_ref, a1_ref):
    B = b_ref[...]
    x0 = x0_ref[...]

    x1 = jax.lax.dot_general(B, x0, _CONTRACT0,
                             preferred_element_type=_F32)
    x0a = jnp.dot(B, jnp.dot(x1, w1_ref[...],
                             preferred_element_type=_F32),
                  preferred_element_type=_F32)
    x1a = jax.lax.dot_general(B, x0a, _CONTRACT0,
                              preferred_element_type=_F32)
    x0b = jnp.dot(B, jnp.dot(x1a, w2_ref[...],
                             preferred_element_type=_F32),
                  preferred_element_type=_F32)

    wm1 = wm1_ref[...]
    a1_ref[...] = jnp.dot(x0b, wm1[:_D, :],
                          preferred_element_type=_F32)
    C1b = (jnp.dot(x1a, wm1[_D:, :], preferred_element_type=_F32)
           + bm1_ref[...][None, :])
    c1rows = [
        jnp.concatenate([C1b[_PACK * q + j:_PACK * q + j + 1, :]
                         for j in range(_PACK)], axis=1)
        for q in range(_NG)
    ]

    ri = jax.lax.broadcasted_iota(jnp.int32, (_KP, _KP), 0)
    ci = jax.lax.broadcasted_iota(jnp.int32, (_KP, _KP), 1)
    w2x = jnp.where((ri // _D) == (ci // _D),
                    jnp.tile(wm2_ref[...], (_PACK, _PACK)),
                    0.0).astype(_BF16)
    ri3 = jax.lax.broadcasted_iota(jnp.int32, (_KP, _PACK), 0)
    ci3 = jax.lax.broadcasted_iota(jnp.int32, (_KP, _PACK), 1)
    w3x = jnp.where((ri3 // _D) == ci3,
                    jnp.tile(wm3_ref[...], (_PACK, _PACK)),
                    0.0).astype(_BF16)
    bm2t = jnp.tile(bm2_ref[...][None, :], (1, _PACK))
    bm3s = bm3_ref[0]

    def body(i, carry):
        base = i * _BN
        a1 = a1_ref[pl.ds(base, _BN), :]
        a1t = jnp.tile(a1, (1, _PACK))
        cols = []
        for q in range(_NG):
            h1b = (a1t + c1rows[q]).astype(_BF16)
            h2 = jnp.dot(h1b, w2x, preferred_element_type=_F32) + bm2t
            h2b = h2.astype(_BF16)
            cols.append(jnp.dot(h2b, w3x,
                                preferred_element_type=_F32))
        h3 = jnp.concatenate(cols, axis=1) + bm3s
        bblk = b_ref[pl.ds(base, _BN), :]
        out_ref[pl.ds(base, _BN), :] = jnp.where(bblk != 0, h3, 0.0)
        return carry

    jax.lax.fori_loop(0, _NB, body, 0)


def kernel(x_0, incidence_1, W1, W2, Wm1, bm1, Wm2, bm2, Wm3, bm3):
    n, m = incidence_1.shape
    return pl.pallas_call(
        _fused_kernel,
        out_shape=jax.ShapeDtypeStruct((n, m), jnp.float32),
        scratch_shapes=[pltpu.VMEM((n, x_0.shape[1]), jnp.float32)],
    )(x_0, incidence_1, W1, W2, Wm1, bm1, Wm2, bm2, Wm3, bm3)


# speedup vs baseline: 1.0449x; 1.0449x over previous
import jax
import jax.numpy as jnp
from jax.experimental import pallas as pl
from jax.experimental.pallas import tpu as pltpu

_N, _M, _D = 10000, 64, 32
_BN = 1000
_NB = _N // _BN
_PACK = 8
_KP = _PACK * _D
_NG = _M // _PACK

_F32 = jnp.float32
_BF16 = jnp.bfloat16
_CONTRACT0 = (((0,), (0,)), ((), ()))


def _bf(x):
    return x.astype(_BF16)


def _fused_kernel(x0_ref, b_ref, w1_ref, w2_ref, wm1_ref, bm1_ref,
                  wm2_ref, bm2_ref, wm3_ref, bm3_ref, out_ref, a1_ref):
    Bb = _bf(b_ref[...])

    x1 = jax.lax.dot_general(Bb, _bf(x0_ref[...]), _CONTRACT0,
                             preferred_element_type=_F32)
    z1 = jnp.dot(_bf(x1), _bf(w1_ref[...]), preferred_element_type=_F32)
    x0a = jnp.dot(Bb, _bf(z1), preferred_element_type=_F32)
    x1a = jax.lax.dot_general(Bb, _bf(x0a), _CONTRACT0,
                              preferred_element_type=_F32)
    z2 = jnp.dot(_bf(x1a), _bf(w2_ref[...]), preferred_element_type=_F32)
    x0b = jnp.dot(Bb, _bf(z2), preferred_element_type=_F32)

    wm1b = _bf(wm1_ref[...])
    a1_ref[...] = jnp.dot(_bf(x0b), wm1b[:_D, :],
                          preferred_element_type=_F32)
    C1b = (jnp.dot(_bf(x1a), wm1b[_D:, :], preferred_element_type=_F32)
           + bm1_ref[...][None, :])
    c1rows = [
        jnp.concatenate([C1b[_PACK * q + j:_PACK * q + j + 1, :]
                         for j in range(_PACK)], axis=1)
        for q in range(_NG)
    ]

    ri = jax.lax.broadcasted_iota(jnp.int32, (_KP, _KP), 0)
    ci = jax.lax.broadcasted_iota(jnp.int32, (_KP, _KP), 1)
    w2x = jnp.where((ri // _D) == (ci // _D),
                    jnp.tile(wm2_ref[...], (_PACK, _PACK)),
                    0.0).astype(_BF16)
    ri3 = jax.lax.broadcasted_iota(jnp.int32, (_KP, _PACK), 0)
    ci3 = jax.lax.broadcasted_iota(jnp.int32, (_KP, _PACK), 1)
    w3x = jnp.where((ri3 // _D) == ci3,
                    jnp.tile(wm3_ref[...], (_PACK, _PACK)),
                    0.0).astype(_BF16)
    bm2t = jnp.tile(bm2_ref[...][None, :], (1, _PACK))
    bm3s = bm3_ref[0]

    def body(i, carry):
        base = i * _BN
        a1 = a1_ref[pl.ds(base, _BN), :]
        a1t = jnp.tile(a1, (1, _PACK))
        h2bs = []
        for q in range(_NG):
            h1b = _bf(a1t + c1rows[q])
            h2 = jnp.dot(h1b, w2x, preferred_element_type=_F32) + bm2t
            h2bs.append(_bf(h2))
        cols = [jnp.dot(h2b, w3x, preferred_element_type=_F32)
                for h2b in h2bs]
        h3 = jnp.concatenate(cols, axis=1) + bm3s
        bblk = b_ref[pl.ds(base, _BN), :]
        out_ref[pl.ds(base, _BN), :] = jnp.where(bblk != 0, h3, 0.0)
        return carry

    jax.lax.fori_loop(0, _NB, body, 0)


def kernel(x_0, incidence_1, W1, W2, Wm1, bm1, Wm2, bm2, Wm3, bm3):
    n, m = incidence_1.shape
    return pl.pallas_call(
        _fused_kernel,
        out_shape=jax.ShapeDtypeStruct((n, m), jnp.float32),
        scratch_shapes=[pltpu.VMEM((n, x_0.shape[1]), jnp.float32)],
    )(x_0, incidence_1, W1, W2, Wm1, bm1, Wm2, bm2, Wm3, bm3)


# speedup vs baseline: 1.0725x; 1.0264x over previous
import jax
import jax.numpy as jnp
from jax.experimental import pallas as pl
from jax.experimental.pallas import tpu as pltpu

_N, _M, _D = 10000, 64, 32
_BN = 2000
_NB = _N // _BN
_PACK = 8
_KP = _PACK * _D
_NG = _M // _PACK

_F32 = jnp.float32
_BF16 = jnp.bfloat16
_CONTRACT0 = (((0,), (0,)), ((), ()))


def _bf(x):
    return x.astype(_BF16)


def _fused_kernel(x0_ref, b_ref, w1_ref, w2_ref, wm1_ref, bm1_ref,
                  wm2_ref, bm2_ref, wm3_ref, bm3_ref, out_ref, a1_ref):
    Bb = _bf(b_ref[...])

    x1 = jax.lax.dot_general(Bb, _bf(x0_ref[...]), _CONTRACT0,
                             preferred_element_type=_F32)
    z1 = jnp.dot(_bf(x1), _bf(w1_ref[...]), preferred_element_type=_F32)
    x0a = jnp.dot(Bb, _bf(z1), preferred_element_type=_F32)
    x1a = jax.lax.dot_general(Bb, _bf(x0a), _CONTRACT0,
                              preferred_element_type=_F32)
    z2 = jnp.dot(_bf(x1a), _bf(w2_ref[...]), preferred_element_type=_F32)
    x0b = jnp.dot(Bb, _bf(z2), preferred_element_type=_F32)

    wm1b = _bf(wm1_ref[...])
    a1_ref[...] = jnp.dot(_bf(x0b), wm1b[:_D, :],
                          preferred_element_type=_F32)
    C1b = (jnp.dot(_bf(x1a), wm1b[_D:, :], preferred_element_type=_F32)
           + bm1_ref[...][None, :])
    c1rows = [
        jnp.concatenate([C1b[_PACK * q + j:_PACK * q + j + 1, :]
                         for j in range(_PACK)], axis=1)
        for q in range(_NG)
    ]

    ri = jax.lax.broadcasted_iota(jnp.int32, (_KP, _KP), 0)
    ci = jax.lax.broadcasted_iota(jnp.int32, (_KP, _KP), 1)
    w2x = jnp.where((ri // _D) == (ci // _D),
                    jnp.tile(wm2_ref[...], (_PACK, _PACK)),
                    0.0).astype(_BF16)
    ri3 = jax.lax.broadcasted_iota(jnp.int32, (_KP, _PACK), 0)
    ci3 = jax.lax.broadcasted_iota(jnp.int32, (_KP, _PACK), 1)
    w3x = jnp.where((ri3 // _D) == ci3,
                    jnp.tile(wm3_ref[...], (_PACK, _PACK)),
                    0.0).astype(_BF16)
    bm2t = jnp.tile(bm2_ref[...][None, :], (1, _PACK))
    bm3s = bm3_ref[0]

    def body(i, carry):
        base = i * _BN
        a1 = a1_ref[pl.ds(base, _BN), :]
        a1t = jnp.tile(a1, (1, _PACK))
        h2bs = []
        for q in range(_NG):
            h1b = _bf(a1t + c1rows[q])
            h2 = jnp.dot(h1b, w2x, preferred_element_type=_F32) + bm2t
            h2bs.append(_bf(h2))
        cols = [jnp.dot(h2b, w3x, preferred_element_type=_F32)
                for h2b in h2bs]
        h3 = jnp.concatenate(cols, axis=1) + bm3s
        bblk = b_ref[pl.ds(base, _BN), :]
        out_ref[pl.ds(base, _BN), :] = jnp.where(bblk != 0, h3, 0.0)
        return carry

    jax.lax.fori_loop(0, _NB, body, 0)


def kernel(x_0, incidence_1, W1, W2, Wm1, bm1, Wm2, bm2, Wm3, bm3):
    n, m = incidence_1.shape
    return pl.pallas_call(
        _fused_kernel,
        out_shape=jax.ShapeDtypeStruct((n, m), jnp.float32),
        scratch_shapes=[pltpu.VMEM((n, x_0.shape[1]), jnp.float32)],
    )(x_0, incidence_1, W1, W2, Wm1, bm1, Wm2, bm2, Wm3, bm3)


# speedup vs baseline: 1.1127x; 1.0374x over previous
import jax
import jax.numpy as jnp
from jax.experimental import pallas as pl
from jax.experimental.pallas import tpu as pltpu

_N, _M, _D = 10000, 64, 32
_BN = 2000
_NB = _N // _BN
_PACK = 8
_KP = _PACK * _D
_NG = _M // _PACK

_F32 = jnp.float32
_BF16 = jnp.bfloat16
_CONTRACT0 = (((0,), (0,)), ((), ()))


def _bf(x):
    return x.astype(_BF16)


def _fused_kernel(x0_ref, b_ref, w1_ref, w2_ref, wm1_ref, bm1_ref,
                  wm2_ref, bm2_ref, wm3_ref, bm3_ref, out_ref, a1_ref):
    Bb = b_ref[...]

    x1 = jax.lax.dot_general(Bb, x0_ref[...], _CONTRACT0,
                             preferred_element_type=_F32)
    z1 = jnp.dot(_bf(x1), _bf(w1_ref[...]), preferred_element_type=_F32)
    x0a = jnp.dot(Bb, _bf(z1), preferred_element_type=_F32)
    x1a = jax.lax.dot_general(Bb, _bf(x0a), _CONTRACT0,
                              preferred_element_type=_F32)
    z2 = jnp.dot(_bf(x1a), _bf(w2_ref[...]), preferred_element_type=_F32)
    x0b = jnp.dot(Bb, _bf(z2), preferred_element_type=_F32)

    wm1b = _bf(wm1_ref[...])
    a1_ref[...] = jnp.dot(_bf(x0b), wm1b[:_D, :],
                          preferred_element_type=_F32)
    C1b = (jnp.dot(_bf(x1a), wm1b[_D:, :], preferred_element_type=_F32)
           + bm1_ref[...][None, :])
    c1rows = [
        jnp.concatenate([C1b[_PACK * q + j:_PACK * q + j + 1, :]
                         for j in range(_PACK)], axis=1)
        for q in range(_NG)
    ]

    ri = jax.lax.broadcasted_iota(jnp.int32, (_KP, _KP), 0)
    ci = jax.lax.broadcasted_iota(jnp.int32, (_KP, _KP), 1)
    w2x = jnp.where((ri // _D) == (ci // _D),
                    jnp.tile(wm2_ref[...], (_PACK, _PACK)),
                    0.0).astype(_BF16)
    ri3 = jax.lax.broadcasted_iota(jnp.int32, (_KP, _PACK), 0)
    ci3 = jax.lax.broadcasted_iota(jnp.int32, (_KP, _PACK), 1)
    w3x = jnp.where((ri3 // _D) == ci3,
                    jnp.tile(wm3_ref[...], (_PACK, _PACK)),
                    0.0).astype(_BF16)
    bm2t = jnp.tile(bm2_ref[...][None, :], (1, _PACK))
    bm3s = bm3_ref[0]

    def body(i, carry):
        base = i * _BN
        a1 = a1_ref[pl.ds(base, _BN), :]
        a1t = jnp.tile(a1, (1, _PACK))
        h2bs = []
        for q in range(_NG):
            h1b = _bf(a1t + c1rows[q])
            h2 = jnp.dot(h1b, w2x, preferred_element_type=_F32) + bm2t
            h2bs.append(_bf(h2))
        cols = [jnp.dot(h2b, w3x, preferred_element_type=_F32)
                for h2b in h2bs]
        h3 = jnp.concatenate(cols, axis=1) + bm3s
        bblk = b_ref[pl.ds(base, _BN), :]
        out_ref[pl.ds(base, _BN), :] = jnp.where(bblk != 0, h3, 0.0)
        return carry

    jax.lax.fori_loop(0, _NB, body, 0)


def kernel(x_0, incidence_1, W1, W2, Wm1, bm1, Wm2, bm2, Wm3, bm3):
    n, m = incidence_1.shape
    return pl.pallas_call(
        _fused_kernel,
        out_shape=jax.ShapeDtypeStruct((n, m), jnp.float32),
        scratch_shapes=[pltpu.VMEM((n, x_0.shape[1]), jnp.float32)],
    )(x_0.astype(jnp.bfloat16), incidence_1.astype(jnp.bfloat16),
      W1, W2, Wm1, bm1, Wm2, bm2, Wm3, bm3)
